# baseline (speedup 1.0000x reference)
import functools

import jax
import jax.numpy as jnp
from jax import lax
from jax.experimental import pallas as pl
from jax.experimental.pallas import tpu as pltpu
from jax.experimental.pallas import tpu_sc as plsc

BATCH = 16384
DIM = 64
IDX_ROWS = 128
IDX_COLS = 128
ROWS_PER_W = 4

MLP_BLOCK = 2048


def _sc_gather_body(uidx_hbm, iidx_hbm, ue_hbm, ie_hbm, u_out, i_out,
                    uidx_v, iidx_v, urows_v, irows_v, sem):
    nc = 2
    wid = lax.axis_index("s") * nc + lax.axis_index("c")
    base = wid * ROWS_PER_W
    pltpu.sync_copy(uidx_hbm.at[pl.ds(base, ROWS_PER_W)], uidx_v)
    pltpu.sync_copy(iidx_hbm.at[pl.ds(base, ROWS_PER_W)], iidx_v)
    copies = []
    for j in range(ROWS_PER_W):
        copies.append(pltpu.async_copy(ue_hbm.at[uidx_v.at[j]], urows_v.at[j], sem))
        copies.append(pltpu.async_copy(ie_hbm.at[iidx_v.at[j]], irows_v.at[j], sem))
    for c in copies:
        c.wait()
    pltpu.sync_copy(urows_v, u_out.at[pl.ds(base, ROWS_PER_W)])
    pltpu.sync_copy(irows_v, i_out.at[pl.ds(base, ROWS_PER_W)])


def _sc_gather(users2d, items2d, user_embedding, item_embedding):
    mesh = plsc.VectorSubcoreMesh(core_axis_name="c", subcore_axis_name="s")
    kern = functools.partial(
        pl.kernel,
        mesh=mesh,
        out_type=[
            jax.ShapeDtypeStruct((IDX_ROWS, IDX_COLS, DIM), jnp.float32),
            jax.ShapeDtypeStruct((IDX_ROWS, IDX_COLS, DIM), jnp.float32),
        ],
        scratch_types=[
            pltpu.VMEM((ROWS_PER_W, IDX_COLS), jnp.int32),
            pltpu.VMEM((ROWS_PER_W, IDX_COLS), jnp.int32),
            pltpu.VMEM((ROWS_PER_W, IDX_COLS, DIM), jnp.float32),
            pltpu.VMEM((ROWS_PER_W, IDX_COLS, DIM), jnp.float32),
            pltpu.SemaphoreType.DMA,
        ],
        compiler_params=pltpu.CompilerParams(use_tc_tiling_on_sc=False),
    )(_sc_gather_body)
    return kern(users2d, items2d, user_embedding, item_embedding)


def _mlp_body(u_ref, i_ref, w1a_ref, w1b_ref, b1_ref, w2_ref, b2_ref,
              w3_ref, b3_ref, o_ref):
    h = jnp.dot(u_ref[...], w1a_ref[...], preferred_element_type=jnp.float32)
    h = h + jnp.dot(i_ref[...], w1b_ref[...], preferred_element_type=jnp.float32)
    h = jnp.maximum(h + b1_ref[...], 0.0)
    h = jnp.dot(h, w2_ref[...], preferred_element_type=jnp.float32) + b2_ref[...]
    h = jnp.maximum(h, 0.0)
    o = jnp.dot(h, w3_ref[...], preferred_element_type=jnp.float32) + b3_ref[...]
    o_ref[...] = 1.0 / (1.0 + jnp.exp(-o))


def _mlp(u_rows, i_rows, W1a, W1b, b1, W2, b2, W3, b3):
    n_blocks = BATCH // MLP_BLOCK
    return pl.pallas_call(
        _mlp_body,
        grid=(n_blocks,),
        in_specs=[
            pl.BlockSpec((MLP_BLOCK, DIM), lambda i: (i, 0)),
            pl.BlockSpec((MLP_BLOCK, DIM), lambda i: (i, 0)),
            pl.BlockSpec((DIM, 32), lambda i: (0, 0)),
            pl.BlockSpec((DIM, 32), lambda i: (0, 0)),
            pl.BlockSpec((1, 32), lambda i: (0, 0)),
            pl.BlockSpec((32, 16), lambda i: (0, 0)),
            pl.BlockSpec((1, 16), lambda i: (0, 0)),
            pl.BlockSpec((16, 1), lambda i: (0, 0)),
            pl.BlockSpec((1, 1), lambda i: (0, 0)),
        ],
        out_specs=pl.BlockSpec((MLP_BLOCK, 1), lambda i: (i, 0)),
        out_shape=jax.ShapeDtypeStruct((BATCH, 1), jnp.float32),
    )(u_rows, i_rows, W1a, W1b, b1, W2, b2, W3, b3)


def kernel(users, items, user_embedding, item_embedding, W1, b1, W2, b2, W3, b3):
    users2d = users.astype(jnp.int32).reshape(IDX_ROWS, IDX_COLS)
    items2d = items.astype(jnp.int32).reshape(IDX_ROWS, IDX_COLS)
    u_rows, i_rows = _sc_gather(users2d, items2d, user_embedding, item_embedding)
    u_rows = u_rows.reshape(BATCH, DIM)
    i_rows = i_rows.reshape(BATCH, DIM)
    out = _mlp(
        u_rows, i_rows,
        W1[:DIM], W1[DIM:],
        b1.reshape(1, 32),
        W2, b2.reshape(1, 16),
        W3, b3.reshape(1, 1),
    )
    return (out, out)
